# baseline (speedup 1.0000x reference)
import jax
import jax.numpy as jnp
from jax.experimental import pallas as pl
from jax.experimental.pallas import tpu as pltpu

_SEQ_BLK = 512
_SUB = 8


def _scan_body(b_ref, c_ref, a_ref, y_ref, h_ref):
    s = pl.program_id(1)

    @pl.when(s == 0)
    def _():
        h_ref[...] = jnp.zeros_like(h_ref)

    iota = jax.lax.broadcasted_iota(jnp.int32, (1, _SUB, 1), 1)

    def chunk(c, h):
        r = pl.ds(c * _SUB, _SUB)
        A = a_ref[:, r, :]
        Bv = b_ref[:, r, :]
        for k in (1, 2, 4):
            mask = iota < k
            A_sh = jnp.where(mask, 1.0, jnp.roll(A, k, axis=1))
            B_sh = jnp.where(mask, 0.0, jnp.roll(Bv, k, axis=1))
            Bv = A * B_sh + Bv
            A = A * A_sh
        hr = A * h + Bv
        y_ref[:, r, :] = c_ref[:, r, :] * hr
        return hr[:, _SUB - 1 : _SUB, :]

    h = jax.lax.fori_loop(0, _SEQ_BLK // _SUB, chunk, h_ref[...])
    h_ref[...] = h


@jax.jit
def kernel(x, B, C, A):
    del x
    batch, seq_len, state_dim = B.shape
    bb = 1

    blk = (bb, _SEQ_BLK, state_dim)
    spec = pl.BlockSpec(blk, lambda p, s: (p, s, 0))

    return pl.pallas_call(
        _scan_body,
        grid=(batch // bb, seq_len // _SEQ_BLK),
        in_specs=[spec, spec, spec],
        out_specs=spec,
        out_shape=jax.ShapeDtypeStruct((batch, seq_len, state_dim), B.dtype),
        scratch_shapes=[pltpu.VMEM((bb, 1, state_dim), jnp.float32)],
        compiler_params=pltpu.CompilerParams(
            dimension_semantics=("parallel", "arbitrary"),
        ),
    )(B, C, A)


# speedup vs baseline: 1.0894x; 1.0894x over previous
import jax
import jax.numpy as jnp
from jax.experimental import pallas as pl
from jax.experimental.pallas import tpu as pltpu

_SEQ_BLK = 512
_SUB = 8


def _scan_body(b_ref, c_ref, a_ref, y_ref, h_ref):
    s = pl.program_id(1)

    @pl.when(s == 0)
    def _():
        h_ref[...] = jnp.zeros_like(h_ref)

    iota = jax.lax.broadcasted_iota(jnp.int32, (1, _SUB, 1), 1)

    def chunk(c, h):
        r = pl.ds(c * _SUB, _SUB)
        A = a_ref[:, r, :]
        Bv = b_ref[:, r, :]
        for k in (1, 2, 4):
            mask = iota < k
            A_sh = jnp.where(mask, 1.0, jnp.roll(A, k, axis=1))
            B_sh = jnp.where(mask, 0.0, jnp.roll(Bv, k, axis=1))
            Bv = A * B_sh + Bv
            A = A * A_sh
        hr = A * h + Bv
        y_ref[:, r, :] = c_ref[:, r, :] * hr
        return hr[:, _SUB - 1 : _SUB, :]

    def chunk2(i, h):
        h = chunk(2 * i, h)
        return chunk(2 * i + 1, h)

    h = jax.lax.fori_loop(0, _SEQ_BLK // (2 * _SUB), chunk2, h_ref[...])
    h_ref[...] = h


@jax.jit
def kernel(x, B, C, A):
    del x
    batch, seq_len, state_dim = B.shape
    bb = 2

    blk = (bb, _SEQ_BLK, state_dim)
    spec = pl.BlockSpec(blk, lambda p, s: (p, s, 0))

    return pl.pallas_call(
        _scan_body,
        grid=(batch // bb, seq_len // _SEQ_BLK),
        in_specs=[spec, spec, spec],
        out_specs=spec,
        out_shape=jax.ShapeDtypeStruct((batch, seq_len, state_dim), B.dtype),
        scratch_shapes=[pltpu.VMEM((bb, 1, state_dim), jnp.float32)],
        compiler_params=pltpu.CompilerParams(
            dimension_semantics=("parallel", "arbitrary"),
        ),
    )(B, C, A)


# speedup vs baseline: 1.0924x; 1.0028x over previous
import jax
import jax.numpy as jnp
from jax.experimental import pallas as pl
from jax.experimental.pallas import tpu as pltpu

_SEQ_BLK = 512
_SUB = 8


def _scan_body(b_ref, c_ref, a_ref, y_ref, h_ref):
    s = pl.program_id(1)

    @pl.when(s == 0)
    def _():
        h_ref[...] = jnp.zeros_like(h_ref)

    iota = jax.lax.broadcasted_iota(jnp.int32, (1, _SUB, 1), 1)

    def chunk(c, h):
        r = pl.ds(c * _SUB, _SUB)
        A = a_ref[:, r, :]
        Bv = b_ref[:, r, :]
        for k in (1, 2, 4):
            mask = iota < k
            A_sh = jnp.where(mask, 1.0, jnp.roll(A, k, axis=1))
            B_sh = jnp.where(mask, 0.0, jnp.roll(Bv, k, axis=1))
            Bv = A * B_sh + Bv
            A = A * A_sh
        hr = A * h + Bv
        y_ref[:, r, :] = c_ref[:, r, :] * hr
        return hr[:, _SUB - 1 : _SUB, :]

    h = jax.lax.fori_loop(0, _SEQ_BLK // _SUB, chunk, h_ref[...])
    h_ref[...] = h


@jax.jit
def kernel(x, B, C, A):
    del x
    batch, seq_len, state_dim = B.shape
    bb = 2

    blk = (bb, _SEQ_BLK, state_dim)
    spec = pl.BlockSpec(blk, lambda p, s: (p, s, 0))

    return pl.pallas_call(
        _scan_body,
        grid=(batch // bb, seq_len // _SEQ_BLK),
        in_specs=[spec, spec, spec],
        out_specs=spec,
        out_shape=jax.ShapeDtypeStruct((batch, seq_len, state_dim), B.dtype),
        scratch_shapes=[pltpu.VMEM((bb, 1, state_dim), jnp.float32)],
        compiler_params=pltpu.CompilerParams(
            dimension_semantics=("parallel", "arbitrary"),
        ),
    )(B, C, A)
